# baseline (speedup 1.0000x reference)
import jax
import jax.numpy as jnp
from jax.experimental import pallas as pl
from jax.experimental.pallas import tpu as pltpu

_ROWS, _COLS = 50176, 768
_BLK = 3584
_N = _ROWS // _BLK
_NBUF = 4


def _dma_body(x_hbm, o_hbm, bufs, in_sems, out_sems):
    def in_cp(k):
        return pltpu.make_async_copy(
            x_hbm.at[pl.ds(k * _BLK, _BLK)], bufs.at[k % _NBUF],
            in_sems.at[k % _NBUF],
        )

    def out_cp(k):
        return pltpu.make_async_copy(
            bufs.at[k % _NBUF], o_hbm.at[pl.ds(k * _BLK, _BLK)],
            out_sems.at[k % _NBUF],
        )

    in_cp(0).start()
    for k in range(_N):
        in_cp(k).wait()
        if k + 1 < _N:
            if k + 1 - _NBUF >= 0:
                out_cp(k + 1 - _NBUF).wait()
            in_cp(k + 1).start()
        out_cp(k).start()
    for k in range(max(0, _N - _NBUF), _N):
        out_cp(k).wait()


def kernel(x, center):
    del center
    flat = x.transpose(2, 3, 0, 1).reshape(_ROWS, _COLS)
    yt = pl.pallas_call(
        _dma_body,
        in_specs=[pl.BlockSpec(memory_space=pltpu.MemorySpace.HBM)],
        out_specs=pl.BlockSpec(memory_space=pltpu.MemorySpace.HBM),
        out_shape=jax.ShapeDtypeStruct((_ROWS, _COLS), x.dtype),
        scratch_shapes=[
            pltpu.VMEM((_NBUF, _BLK, _COLS), jnp.float32),
            pltpu.SemaphoreType.DMA((_NBUF,)),
            pltpu.SemaphoreType.DMA((_NBUF,)),
        ],
    )(flat)
    return yt.reshape(14, 14, 256, 768).transpose(2, 3, 0, 1)


# speedup vs baseline: 1.1872x; 1.1872x over previous
import jax
import jax.numpy as jnp
from jax.experimental import pallas as pl
from jax.experimental.pallas import tpu as pltpu

_ROWS, _COLS = 50176, 768
_BLK = 3584


def _copy_body(x_ref, o_ref):
    o_ref[...] = x_ref[...]


def kernel(x, center):
    del center
    flat = x.transpose(2, 3, 0, 1).reshape(_ROWS, _COLS)
    yt = pl.pallas_call(
        _copy_body,
        grid=(_ROWS // _BLK,),
        in_specs=[pl.BlockSpec((_BLK, _COLS), lambda i: (i, 0))],
        out_specs=pl.BlockSpec((_BLK, _COLS), lambda i: (i, 0)),
        out_shape=jax.ShapeDtypeStruct((_ROWS, _COLS), x.dtype),
        compiler_params=pltpu.CompilerParams(
            dimension_semantics=("parallel",),
        ),
    )(flat)
    return yt.reshape(14, 14, 256, 768).transpose(2, 3, 0, 1)


# speedup vs baseline: 1.1880x; 1.0007x over previous
import jax
import jax.numpy as jnp
from jax.experimental import pallas as pl
from jax.experimental.pallas import tpu as pltpu

_ROWS, _COLS = 50176, 768
_BLK = 3584


def _copy_body(x_ref, o_ref):
    o_ref[...] = x_ref[...]


def kernel(x, center):
    del center
    flat = x.transpose(2, 3, 0, 1).reshape(_ROWS, _COLS)
    yt = pl.pallas_call(
        _copy_body,
        grid=(_ROWS // _BLK,),
        in_specs=[pl.BlockSpec((_BLK, _COLS), lambda i: (i, 0))],
        out_specs=pl.BlockSpec((_BLK, _COLS), lambda i: (i, 0)),
        out_shape=jax.ShapeDtypeStruct((_ROWS, _COLS), x.dtype),
        compiler_params=pltpu.CompilerParams(
            dimension_semantics=("arbitrary",),
        ),
    )(flat)
    return yt.reshape(14, 14, 256, 768).transpose(2, 3, 0, 1)
